# Initial kernel scaffold; baseline (speedup 1.0000x reference)
import functools

import jax
import jax.numpy as jnp
import numpy as np
from jax import lax
from jax.experimental import pallas as pl
from jax.experimental.pallas import tpu as pltpu
from jax.experimental.pallas import tpu_sc as plsc

EPS = 1e-6
NS = 0.2
KNN = 10
TN = 128
GCH = 128
INV_BN = np.float32(1.0 / np.sqrt(1.0 + 1e-5))


def _knn_body(xf_ref, xt_ref, idx_ref):
    n = xf_ref.shape[2]
    c = xf_ref.shape[3]
    ones = jnp.ones((1, c), jnp.float32)
    acc = None
    sf = None
    st = None
    for a in range(3):
        xf = xf_ref[0, a]
        xt = xt_ref[0, a]
        m = lax.dot_general(xf, xt, (((1,), (1,)), ((), ())),
                            preferred_element_type=jnp.float32)
        acc = m if a == 0 else acc + m
        sfa = jnp.sum(xf * xf, axis=1, keepdims=True)
        sf = sfa if a == 0 else sf + sfa
        sta = lax.dot_general(ones, xt * xt, (((1,), (1,)), ((), ())),
                              preferred_element_type=jnp.float32)
        st = sta if a == 0 else st + sta
    pd = 2.0 * acc - sf - st
    row_iota = lax.broadcasted_iota(jnp.int32, (n, TN), 0)
    neg_inf = jnp.float32(-jnp.inf)
    for j in range(KNN):
        mx = jnp.max(pd, axis=0, keepdims=True)
        cand = jnp.where(pd == mx, row_iota, n)
        idxj = jnp.min(cand, axis=0, keepdims=True)
        idx_ref[0, j:j + 1, :] = idxj
        pd = jnp.where(row_iota == idxj, neg_inf, pd)


def _knn_call(x):
    b, _, n, c = x.shape
    grid = (b, n // TN)
    return pl.pallas_call(
        _knn_body,
        grid=grid,
        in_specs=[
            pl.BlockSpec((1, 3, n, c), lambda bb, t: (bb, 0, 0, 0)),
            pl.BlockSpec((1, 3, TN, c), lambda bb, t: (bb, 0, t, 0)),
        ],
        out_specs=pl.BlockSpec((1, KNN, TN), lambda bb, t: (bb, 0, t)),
        out_shape=jax.ShapeDtypeStruct((b, KNN, n), jnp.int32),
    )(x, x)


def _gather_call(table, flat_idx):
    v, c = table.shape
    r = flat_idx.shape[0]
    info = plsc.get_sparse_core_info()
    nc, nsub = info.num_cores, info.num_subcores
    nw = nc * nsub
    per_w = r // nw
    nch = per_w // GCH
    mesh = plsc.VectorSubcoreMesh(core_axis_name="c", subcore_axis_name="s")

    @functools.partial(
        pl.kernel,
        mesh=mesh,
        out_type=jax.ShapeDtypeStruct((r, c), jnp.float32),
        scratch_types=[
            pltpu.VMEM((GCH,), jnp.int32),
            pltpu.VMEM((GCH, c), jnp.float32),
            pltpu.SemaphoreType.DMA,
        ],
    )
    def gk(tab_hbm, idx_hbm, out_hbm, idx_v, rows_v, sem):
        wid = lax.axis_index("s") * nc + lax.axis_index("c")
        base_w = wid * per_w
        for i in range(nch):
            base = base_w + i * GCH
            pltpu.sync_copy(idx_hbm.at[pl.ds(base, GCH)], idx_v)
            pltpu.async_copy(tab_hbm.at[idx_v], rows_v, sem).wait()
            pltpu.sync_copy(rows_v, out_hbm.at[pl.ds(base, GCH)])

    return gk(table, flat_idx)


def _gather_layer(x, idx):
    b, _, n, c = x.shape
    table = x.reshape(b * 3 * n, c)
    off = (jnp.arange(b, dtype=jnp.int32)[:, None, None] * 3
           + jnp.arange(3, dtype=jnp.int32)[None, :, None]) * n
    fidx = (idx.reshape(b, 1, KNN * n) + off).reshape(-1)
    feat = _gather_call(table, fidx)
    return feat.reshape(b, 3, KNN, n, c)


def _layer_body(feat_ref, x_ref, wfa_ref, wfb_ref, wsa_ref, wsb_ref, wdt_ref,
                out_ref):
    wfa = wfa_ref[...]
    wfb = wfb_ref[...]
    wsa = wsa_ref[...]
    wsb = wsb_ref[...]
    wdt = wdt_ref[...]

    def mm(u, w):
        return lax.dot_general(u, w, (((1,), (0,)), ((), ())),
                               preferred_element_type=jnp.float32)

    xr = [x_ref[0, a] for a in range(3)]
    xnorm = jnp.sqrt(xr[0] * xr[0] + xr[1] * xr[1] + xr[2] * xr[2] + EPS)
    sb = mm(xnorm, wsb)
    pb = [mm(xr[a], wfb) for a in range(3)]

    best_dot = None
    best_f = None
    for j in range(KNN):
        fd = [feat_ref[0, a, j] - xr[a] for a in range(3)]
        nd = jnp.sqrt(fd[0] * fd[0] + fd[1] * fd[1] + fd[2] * fd[2] + EPS)
        sc = jax.nn.sigmoid(mm(nd, wsa) + sb)
        f = [(mm(fd[a], wfa) + pb[a]) * sc for a in range(3)]
        d = [mm(f[a], wdt) for a in range(3)]
        dotj = f[0] * d[0] + f[1] * d[1] + f[2] * d[2]
        if j == 0:
            best_dot = dotj
            best_f = f
        else:
            better = dotj > best_dot
            best_dot = jnp.where(better, dotj, best_dot)
            best_f = [jnp.where(better, f[a], best_f[a]) for a in range(3)]
    for a in range(3):
        out_ref[0, a] = best_f[a]


def _layer_call(feat, x, wfa, wfb, wsa, wsb, wdt):
    b, _, _, n, c = feat.shape
    o = wfa.shape[1]
    grid = (b, n // TN)

    def wspec(w):
        nd = w.ndim
        return pl.BlockSpec(w.shape, lambda bb, t: (0,) * nd)

    return pl.pallas_call(
        _layer_body,
        grid=grid,
        in_specs=[
            pl.BlockSpec((1, 3, KNN, TN, c), lambda bb, t: (bb, 0, 0, t, 0)),
            pl.BlockSpec((1, 3, TN, c), lambda bb, t: (bb, 0, t, 0)),
            wspec(wfa), wspec(wfb), wspec(wsa), wspec(wsb), wspec(wdt),
        ],
        out_specs=pl.BlockSpec((1, 3, TN, o), lambda bb, t: (bb, 0, t, 0)),
        out_shape=jax.ShapeDtypeStruct((b, 3, n, o), jnp.float32),
    )(feat, x, wfa, wfb, wsa, wsb, wdt)


def _edgeconv(x, wf, ws, wd, c_real):
    cpad = x.shape[3]
    o = wf.shape[0]
    wfa = jnp.zeros((cpad, o), jnp.float32).at[:c_real].set(wf[:, :c_real].T)
    wfb = jnp.zeros((cpad, o), jnp.float32).at[:c_real].set(wf[:, c_real:].T)
    wsa = jnp.zeros((cpad, o), jnp.float32).at[:c_real].set(ws[:, :c_real].T)
    wsb = jnp.zeros((cpad, o), jnp.float32).at[:c_real].set(ws[:, c_real:].T)
    wdt = wd.T
    idx = _knn_call(x)
    feat = _gather_layer(x, idx)
    return _layer_call(feat, x, wfa, wfb, wsa, wsb, wdt)


def _leaky_pair(h, wft, wdt, mm):
    p = [mm(h[a], wft) * INV_BN for a in range(3)]
    d = [mm(h[a], wdt) for a in range(3)]
    dot = p[0] * d[0] + p[1] * d[1] + p[2] * d[2]
    dsq = d[0] * d[0] + d[1] * d[1] + d[2] * d[2]
    coef = dot / (dsq + EPS)
    mask = (dot >= 0.0).astype(jnp.float32)
    return [NS * p[a]
            + (1.0 - NS) * (mask * p[a]
                            + (1.0 - mask) * (p[a] - coef * d[a]))
            for a in range(3)]


def _agg_body(h_ref, waggf_ref, waggd_ref, s1f_ref, s1d_ref, s2f_ref, s2d_ref,
              sl_ref, w1r_ref, b1_ref, w2t_ref, b2_ref, w3t_ref, b3_ref,
              out_ref):
    def mm(u, w):
        return lax.dot_general(u, w, (((1,), (0,)), ((), ())),
                               preferred_element_type=jnp.float32)

    h = [h_ref[0, a] for a in range(3)]
    h1 = _leaky_pair(h, waggf_ref[...], waggd_ref[...], mm)
    h2 = []
    for a in range(3):
        mean_a = jnp.mean(h1[a], axis=0, keepdims=True)
        h2.append(jnp.concatenate(
            [h1[a], jnp.broadcast_to(mean_a, h1[a].shape)], axis=1))
    z = _leaky_pair(h2, s1f_ref[...], s1d_ref[...], mm)
    z = _leaky_pair(z, s2f_ref[...], s2d_ref[...], mm)
    z0 = [mm(z[a], sl_ref[...]) for a in range(3)]

    def leaky(v):
        return jnp.where(v >= 0.0, v, NS * v)

    s = b1_ref[...]
    for kk in range(3):
        xs = (h2[0] * z0[0][:, kk:kk + 1]
              + h2[1] * z0[1][:, kk:kk + 1]
              + h2[2] * z0[2][:, kk:kk + 1])
        gmax = jnp.max(xs, axis=0, keepdims=True)
        gmean = jnp.mean(xs, axis=0, keepdims=True)
        s = s + mm(gmax, w1r_ref[kk * 682:(kk + 1) * 682, :])
        s = s + mm(gmean, w1r_ref[2046 + kk * 682:2046 + (kk + 1) * 682, :])
    g = leaky(s * INV_BN)
    g = leaky((mm(g, w2t_ref[...]) + b2_ref[...]) * INV_BN)
    out_ref[...] = mm(g, w3t_ref[...]) + b3_ref[...]


def _agg_call(h, waggf, waggd, s1f, s1d, s2f, s2d, slt, w1r, b1, w2t, b2,
              w3t, b3):
    b, _, n, ch = h.shape
    args = (h, waggf, waggd, s1f, s1d, s2f, s2d, slt, w1r, b1, w2t, b2,
            w3t, b3)

    def wspec(w):
        nd = w.ndim
        return pl.BlockSpec(w.shape, lambda bb: (0,) * nd)

    return pl.pallas_call(
        _agg_body,
        grid=(b,),
        in_specs=[pl.BlockSpec((1, 3, n, ch), lambda bb: (bb, 0, 0, 0))]
                 + [wspec(w) for w in args[1:]],
        out_specs=pl.BlockSpec((1, 1), lambda bb: (bb, 0)),
        out_shape=jax.ShapeDtypeStruct((b, 1), jnp.float32),
    )(*args)


def kernel(x, Wf1, Ws1, Wd1, Wf2, Ws2, Wd2, Wf3, Ws3, Wd3, Wf4, Ws4, Wd4,
           Wf5, Ws5, Wd5, Wf6, Ws6, Wd6, Wagg_f, Wagg_d, std1_f, std1_d,
           std2_f, std2_d, std_lin, W1, b1, W2, b2, W3, b3):
    b, _, n = x.shape
    x0 = jnp.zeros((b, 3, n, 16), jnp.float32).at[:, :, :, 0].set(x)

    x1 = _edgeconv(x0, Wf1, Ws1, Wd1, 1)
    x2 = _edgeconv(x1, Wf2, Ws2, Wd2, 16)
    x3 = _edgeconv(x2, Wf3, Ws3, Wd3, 32)
    x4 = _edgeconv(x3, Wf4, Ws4, Wd4, 64)
    x5 = _edgeconv(x4, Wf5, Ws5, Wd5, 128)
    x6 = _edgeconv(x5, Wf6, Ws6, Wd6, 256)

    h = jnp.concatenate([x1, x2, x3, x4, x5, x6], axis=3)

    w1a = W1[:, :2046].reshape(512, 682, 3).transpose(2, 1, 0).reshape(2046, 512)
    w1b = W1[:, 2046:].reshape(512, 682, 3).transpose(2, 1, 0).reshape(2046, 512)
    w1r = jnp.concatenate([w1a, w1b], axis=0)

    out = _agg_call(h, Wagg_f.T, Wagg_d.T, std1_f.T, std1_d.T, std2_f.T,
                    std2_d.T, std_lin.T, w1r, b1.reshape(1, 512), W2.T,
                    b2.reshape(1, 256), W3.T, b3.reshape(1, 1))
    return out[:, 0]


# speedup vs baseline: 78.5070x; 78.5070x over previous
import functools

import jax
import jax.numpy as jnp
import numpy as np
from jax import lax
from jax.experimental import pallas as pl
from jax.experimental.pallas import tpu as pltpu
from jax.experimental.pallas import tpu_sc as plsc

EPS = 1e-6
NS = 0.2
KNN = 10
TN = 128
GCH = 128
INV_BN = np.float32(1.0 / np.sqrt(1.0 + 1e-5))


def _knn_body(xf_ref, xt_ref, idx_ref):
    n = xf_ref.shape[2]
    c = xf_ref.shape[3]
    ones = jnp.ones((1, c), jnp.float32)
    acc = None
    sf = None
    st = None
    for a in range(3):
        xf = xf_ref[0, a]
        xt = xt_ref[0, a]
        m = lax.dot_general(xf, xt, (((1,), (1,)), ((), ())),
                            preferred_element_type=jnp.float32)
        acc = m if a == 0 else acc + m
        sfa = jnp.sum(xf * xf, axis=1, keepdims=True)
        sf = sfa if a == 0 else sf + sfa
        sta = lax.dot_general(ones, xt * xt, (((1,), (1,)), ((), ())),
                              preferred_element_type=jnp.float32)
        st = sta if a == 0 else st + sta
    pd = 2.0 * acc - sf - st
    row_iota = lax.broadcasted_iota(jnp.int32, (n, TN), 0)
    neg_inf = jnp.float32(-jnp.inf)
    for j in range(KNN):
        mx = jnp.max(pd, axis=0, keepdims=True)
        cand = jnp.where(pd == mx, row_iota, n)
        idxj = jnp.min(cand, axis=0, keepdims=True)
        idx_ref[0, j:j + 1, :] = idxj
        pd = jnp.where(row_iota == idxj, neg_inf, pd)


def _knn_call(x):
    b, _, n, c = x.shape
    grid = (b, n // TN)
    return pl.pallas_call(
        _knn_body,
        grid=grid,
        in_specs=[
            pl.BlockSpec((1, 3, n, c), lambda bb, t: (bb, 0, 0, 0)),
            pl.BlockSpec((1, 3, TN, c), lambda bb, t: (bb, 0, t, 0)),
        ],
        out_specs=pl.BlockSpec((1, KNN, TN), lambda bb, t: (bb, 0, t)),
        out_shape=jax.ShapeDtypeStruct((b, KNN, n), jnp.int32),
    )(x, x)


def _gather_call(table, flat_idx):
    v, c = table.shape
    r = flat_idx.shape[0]
    info = plsc.get_sparse_core_info()
    nc, nsub = info.num_cores, info.num_subcores
    nw = nc * nsub
    per_w = r // nw
    nch = per_w // GCH
    mesh = plsc.VectorSubcoreMesh(core_axis_name="c", subcore_axis_name="s")

    @functools.partial(
        pl.kernel,
        mesh=mesh,
        out_type=jax.ShapeDtypeStruct((r, c), jnp.float32),
        scratch_types=[
            pltpu.VMEM((GCH,), jnp.int32),
            pltpu.VMEM((GCH, c), jnp.float32),
            pltpu.SemaphoreType.DMA,
        ],
    )
    def gk(tab_hbm, idx_hbm, out_hbm, idx_v, rows_v, sem):
        wid = lax.axis_index("s") * nc + lax.axis_index("c")
        base_w = wid * per_w
        for i in range(nch):
            base = base_w + i * GCH
            pltpu.sync_copy(idx_hbm.at[pl.ds(base, GCH)], idx_v)
            pltpu.async_copy(tab_hbm.at[idx_v], rows_v, sem).wait()
            pltpu.sync_copy(rows_v, out_hbm.at[pl.ds(base, GCH)])

    return gk(table, flat_idx)


def _gather_layer(x, idx):
    b, _, n, c = x.shape
    table = x.reshape(b * 3 * n, c)
    off = (jnp.arange(b, dtype=jnp.int32)[:, None, None] * 3
           + jnp.arange(3, dtype=jnp.int32)[None, :, None]) * n
    fidx = (idx.reshape(b, 1, KNN * n) + off).reshape(-1)
    feat = _gather_call(table, fidx)
    return feat.reshape(b, 3, KNN, n, c)


def _layer_body(feat_ref, x_ref, wfa_ref, wfb_ref, wsa_ref, wsb_ref, wdt_ref,
                out_ref):
    wfa = wfa_ref[...]
    wfb = wfb_ref[...]
    wsa = wsa_ref[...]
    wsb = wsb_ref[...]
    wdt = wdt_ref[...]

    def mm(u, w):
        return lax.dot_general(u, w, (((1,), (0,)), ((), ())),
                               preferred_element_type=jnp.float32)

    xr = [x_ref[0, a] for a in range(3)]
    xnorm = jnp.sqrt(xr[0] * xr[0] + xr[1] * xr[1] + xr[2] * xr[2] + EPS)
    sb = mm(xnorm, wsb)
    pb = [mm(xr[a], wfb) for a in range(3)]

    best_dot = None
    best_f = None
    for j in range(KNN):
        fd = [feat_ref[0, a, j] - xr[a] for a in range(3)]
        nd = jnp.sqrt(fd[0] * fd[0] + fd[1] * fd[1] + fd[2] * fd[2] + EPS)
        sc = jax.nn.sigmoid(mm(nd, wsa) + sb)
        f = [(mm(fd[a], wfa) + pb[a]) * sc for a in range(3)]
        d = [mm(f[a], wdt) for a in range(3)]
        dotj = f[0] * d[0] + f[1] * d[1] + f[2] * d[2]
        if j == 0:
            best_dot = dotj
            best_f = f
        else:
            better = dotj > best_dot
            best_dot = jnp.where(better, dotj, best_dot)
            best_f = [jnp.where(better, f[a], best_f[a]) for a in range(3)]
    o = best_f[0].shape[1]
    opad = out_ref.shape[3]
    for a in range(3):
        v = best_f[a]
        if opad > o:
            v = jnp.concatenate(
                [v, jnp.zeros((v.shape[0], opad - o), jnp.float32)], axis=1)
        out_ref[0, a] = v


def _layer_call(feat, x, wfa, wfb, wsa, wsb, wdt):
    b, _, _, n, c = feat.shape
    o = wfa.shape[1]
    opad = max(o, 128)
    grid = (b, n // TN)

    def wspec(w):
        nd = w.ndim
        return pl.BlockSpec(w.shape, lambda bb, t: (0,) * nd)

    return pl.pallas_call(
        _layer_body,
        grid=grid,
        in_specs=[
            pl.BlockSpec((1, 3, KNN, TN, c), lambda bb, t: (bb, 0, 0, t, 0)),
            pl.BlockSpec((1, 3, TN, c), lambda bb, t: (bb, 0, t, 0)),
            wspec(wfa), wspec(wfb), wspec(wsa), wspec(wsb), wspec(wdt),
        ],
        out_specs=pl.BlockSpec((1, 3, TN, opad), lambda bb, t: (bb, 0, t, 0)),
        out_shape=jax.ShapeDtypeStruct((b, 3, n, opad), jnp.float32),
    )(feat, x, wfa, wfb, wsa, wsb, wdt)


def _edgeconv(x, wf, ws, wd, c_real):
    cpad = x.shape[3]
    o = wf.shape[0]
    wfa = jnp.zeros((cpad, o), jnp.float32).at[:c_real].set(wf[:, :c_real].T)
    wfb = jnp.zeros((cpad, o), jnp.float32).at[:c_real].set(wf[:, c_real:].T)
    wsa = jnp.zeros((cpad, o), jnp.float32).at[:c_real].set(ws[:, :c_real].T)
    wsb = jnp.zeros((cpad, o), jnp.float32).at[:c_real].set(ws[:, c_real:].T)
    wdt = wd.T
    idx = _knn_call(x)
    feat = _gather_layer(x, idx)
    return _layer_call(feat, x, wfa, wfb, wsa, wsb, wdt)


def _leaky_pair(h, wft, wdt, mm):
    p = [mm(h[a], wft) * INV_BN for a in range(3)]
    d = [mm(h[a], wdt) for a in range(3)]
    dot = p[0] * d[0] + p[1] * d[1] + p[2] * d[2]
    dsq = d[0] * d[0] + d[1] * d[1] + d[2] * d[2]
    coef = dot / (dsq + EPS)
    mask = (dot >= 0.0).astype(jnp.float32)
    return [NS * p[a]
            + (1.0 - NS) * (mask * p[a]
                            + (1.0 - mask) * (p[a] - coef * d[a]))
            for a in range(3)]


def _agg_body(h_ref, waggf_ref, waggd_ref, s1f_ref, s1d_ref, s2f_ref, s2d_ref,
              sl_ref, w1r_ref, b1_ref, w2t_ref, b2_ref, w3t_ref, b3_ref,
              out_ref):
    def mm(u, w):
        return lax.dot_general(u, w, (((1,), (0,)), ((), ())),
                               preferred_element_type=jnp.float32)

    h = [h_ref[0, a] for a in range(3)]
    h1 = _leaky_pair(h, waggf_ref[...], waggd_ref[...], mm)
    h2 = []
    for a in range(3):
        mean_a = jnp.mean(h1[a], axis=0, keepdims=True)
        h2.append(jnp.concatenate(
            [h1[a], jnp.broadcast_to(mean_a, h1[a].shape)], axis=1))
    z = _leaky_pair(h2, s1f_ref[...], s1d_ref[...], mm)
    z = _leaky_pair(z, s2f_ref[...], s2d_ref[...], mm)
    z0 = [mm(z[a], sl_ref[...]) for a in range(3)]

    def leaky(v):
        return jnp.where(v >= 0.0, v, NS * v)

    s = b1_ref[...]
    for kk in range(3):
        xs = (h2[0] * z0[0][:, kk:kk + 1]
              + h2[1] * z0[1][:, kk:kk + 1]
              + h2[2] * z0[2][:, kk:kk + 1])
        gmax = jnp.max(xs, axis=0, keepdims=True)
        gmean = jnp.mean(xs, axis=0, keepdims=True)
        s = s + mm(gmax, w1r_ref[kk * 682:(kk + 1) * 682, :])
        s = s + mm(gmean, w1r_ref[2046 + kk * 682:2046 + (kk + 1) * 682, :])
    g = leaky(s * INV_BN)
    g = leaky((mm(g, w2t_ref[...]) + b2_ref[...]) * INV_BN)
    out_ref[0] = mm(g, w3t_ref[...]) + b3_ref[...]


def _agg_call(h, waggf, waggd, s1f, s1d, s2f, s2d, slt, w1r, b1, w2t, b2,
              w3t, b3):
    b, _, n, ch = h.shape
    args = (h, waggf, waggd, s1f, s1d, s2f, s2d, slt, w1r, b1, w2t, b2,
            w3t, b3)

    def wspec(w):
        nd = w.ndim
        return pl.BlockSpec(w.shape, lambda bb: (0,) * nd)

    return pl.pallas_call(
        _agg_body,
        grid=(b,),
        in_specs=[pl.BlockSpec((1, 3, n, ch), lambda bb: (bb, 0, 0, 0))]
                 + [wspec(w) for w in args[1:]],
        out_specs=pl.BlockSpec((1, 1, 1), lambda bb: (bb, 0, 0)),
        out_shape=jax.ShapeDtypeStruct((b, 1, 1), jnp.float32),
    )(*args)


def kernel(x, Wf1, Ws1, Wd1, Wf2, Ws2, Wd2, Wf3, Ws3, Wd3, Wf4, Ws4, Wd4,
           Wf5, Ws5, Wd5, Wf6, Ws6, Wd6, Wagg_f, Wagg_d, std1_f, std1_d,
           std2_f, std2_d, std_lin, W1, b1, W2, b2, W3, b3):
    b, _, n = x.shape
    x0 = jnp.zeros((b, 3, n, 128), jnp.float32).at[:, :, :, 0].set(x)

    x1 = _edgeconv(x0, Wf1, Ws1, Wd1, 1)
    x2 = _edgeconv(x1, Wf2, Ws2, Wd2, 16)
    x3 = _edgeconv(x2, Wf3, Ws3, Wd3, 32)
    x4 = _edgeconv(x3, Wf4, Ws4, Wd4, 64)
    x5 = _edgeconv(x4, Wf5, Ws5, Wd5, 128)
    x6 = _edgeconv(x5, Wf6, Ws6, Wd6, 256)

    h = jnp.concatenate(
        [x1[..., :16], x2[..., :32], x3[..., :64], x4, x5, x6],
        axis=3)

    w1a = W1[:, :2046].reshape(512, 682, 3).transpose(2, 1, 0).reshape(2046, 512)
    w1b = W1[:, 2046:].reshape(512, 682, 3).transpose(2, 1, 0).reshape(2046, 512)
    w1r = jnp.concatenate([w1a, w1b], axis=0)

    out = _agg_call(h, Wagg_f.T, Wagg_d.T, std1_f.T, std1_d.T, std2_f.T,
                    std2_d.T, std_lin.T, w1r, b1.reshape(1, 512), W2.T,
                    b2.reshape(1, 256), W3.T, b3.reshape(1, 1))
    return out[:, 0, 0]
